# baseline (speedup 1.0000x reference)
import functools

import jax
import jax.numpy as jnp
from jax import lax
from jax.experimental import pallas as pl
from jax.experimental.pallas import tpu as pltpu
from jax.experimental.pallas import tpu_sc as plsc

PAD = 1
HID = 768
EPS = 1e-12

_GATHER_DNUMS = lax.GatherDimensionNumbers(
    offset_dims=(), collapsed_slice_dims=(0,), start_index_map=(0,))


def _shuffle(v, idx):
    return lax.gather(v, idx[:, None], _GATHER_DNUMS, (1,),
                      mode=lax.GatherScatterMode.PROMISE_IN_BOUNDS)

NC = 2
NS = 16
NW = NC * NS
LANES = 16
NVH = HID // LANES


def _posid_body(ids_ref, out_ref):
    ids = ids_ref[...]
    m = (ids != PAD).astype(jnp.int32)
    acc = m
    s = ids.shape[1]
    k = 1
    while k < s:
        shifted = jnp.concatenate(
            [jnp.zeros(ids.shape[:1] + (k,), jnp.int32), acc[:, :-k]], axis=1
        )
        acc = acc + shifted
        k *= 2
    out_ref[...] = acc * m + PAD


def _sc_body(tpw, chunk, word_hbm, pos_hbm, item_hbm, idw_hbm, idp_hbm,
             idi_hbm, lnw_hbm, lnb_hbm, out_hbm,
             idw_all, idp_all, idi_all, bw0, bp0, bw1, bp1, item_v, wv, bv,
             sem, wsem):
    wid = lax.axis_index("s") * NC + lax.axis_index("c")
    base = wid * tpw
    nchunks = tpw // chunk
    pltpu.sync_copy(lnw_hbm, wv)
    pltpu.sync_copy(lnb_hbm, bv)
    pltpu.sync_copy(item_hbm, item_v)
    pltpu.sync_copy(idw_hbm.at[pl.ds(base, tpw)], idw_all)
    pltpu.sync_copy(idp_hbm.at[pl.ds(base, tpw)], idp_all)
    pltpu.sync_copy(idi_hbm.at[pl.ds(base, tpw)], idi_all.at[pl.ds(0, tpw)])

    half = jnp.float32(0.5)
    three_half = jnp.float32(1.5)
    magic = jnp.int32(0x5F3759DF)
    lane = lax.iota(jnp.int32, LANES)
    perms = [lane ^ k for k in (1, 2, 4, 8)]
    bufs = [(bw0, bp0), (bw1, bp1)]

    def issue(g):
        bw, bp = bufs[g % 2]
        sl = pl.ds(g * chunk, chunk)
        cw = pltpu.async_copy(word_hbm.at[idw_all.at[sl]], bw, sem)
        cp = pltpu.async_copy(pos_hbm.at[idp_all.at[sl]], bp, sem)
        return cw, cp

    pend = issue(0)
    wpend = [None, None]
    for g in range(nchunks):
        pend[0].wait()
        pend[1].wait()
        if g + 1 < nchunks:
            if wpend[(g + 1) % 2] is not None:
                wpend[(g + 1) % 2].wait()
                wpend[(g + 1) % 2] = None
            pend = issue(g + 1)
        bw, bp = bufs[g % 2]

        def ln_token(t, _, bw=bw, bp=bp, goff=g * chunk):
            row = idi_all[pl.ds(goff + t, LANES)][0]
            s = jnp.zeros((LANES,), jnp.float32)
            ss = jnp.zeros((LANES,), jnp.float32)
            for i in range(NVH):
                sl = pl.ds(i * LANES, LANES)
                x = bw[t, sl] + bp[t, sl] + item_v[row, sl]
                bw[t, sl] = x
                s = s + x
                ss = ss + x * x
            for p in perms:
                s = s + _shuffle(s, p)
                ss = ss + _shuffle(ss, p)
            mu_v = s * (1.0 / HID)
            vv = ss * (1.0 / HID) - mu_v * mu_v + EPS
            bits = lax.bitcast_convert_type(vv, jnp.int32)
            y = lax.bitcast_convert_type(magic - (bits >> 1), jnp.float32)
            for _it in range(3):
                y = y * (three_half - half * vv * y * y)
            for i in range(NVH):
                sl = pl.ds(i * LANES, LANES)
                x = bw[t, sl]
                bw[t, sl] = (x - mu_v) * y * wv[sl] + bv[sl]
            return 0

        lax.fori_loop(0, chunk, ln_token, 0)
        wpend[g % 2] = pltpu.async_copy(
            bw, out_hbm.at[pl.ds(base + g * chunk, chunk)], wsem)
    for w in wpend:
        if w is not None:
            w.wait()


def kernel(input_ids, item_position_ids, word_embeddings, position_embeddings,
           item_position_embeddings, ln_weight, ln_bias):
    b, s = input_ids.shape
    n = b * s
    tpw = n // NW
    chunk = 32

    position_ids = pl.pallas_call(
        _posid_body,
        out_shape=jax.ShapeDtypeStruct((b, s), jnp.int32),
    )(input_ids.astype(jnp.int32))

    mesh = plsc.VectorSubcoreMesh(core_axis_name="c", subcore_axis_name="s")
    sc = pl.kernel(
        functools.partial(_sc_body, tpw, chunk),
        out_type=jax.ShapeDtypeStruct((n, HID), jnp.float32),
        mesh=mesh,
        scratch_types=[
            pltpu.VMEM((tpw,), jnp.int32),
            pltpu.VMEM((tpw,), jnp.int32),
            pltpu.VMEM((tpw + LANES,), jnp.int32),
            pltpu.VMEM((chunk, HID), jnp.float32),
            pltpu.VMEM((chunk, HID), jnp.float32),
            pltpu.VMEM((chunk, HID), jnp.float32),
            pltpu.VMEM((chunk, HID), jnp.float32),
            pltpu.VMEM(item_position_embeddings.shape, jnp.float32),
            pltpu.VMEM((HID,), jnp.float32),
            pltpu.VMEM((HID,), jnp.float32),
            pltpu.SemaphoreType.DMA,
            pltpu.SemaphoreType.DMA,
        ],
    )
    out = sc(
        word_embeddings,
        position_embeddings,
        item_position_embeddings,
        input_ids.reshape(n).astype(jnp.int32),
        position_ids.reshape(n),
        item_position_ids.reshape(n).astype(jnp.int32),
        ln_weight,
        ln_bias,
    )
    return out.reshape(b, s, HID)


# speedup vs baseline: 2.0011x; 2.0011x over previous
import functools

import jax
import jax.numpy as jnp
from jax import lax
from jax.experimental import pallas as pl
from jax.experimental.pallas import tpu as pltpu
from jax.experimental.pallas import tpu_sc as plsc

PAD = 1
HID = 768
EPS = 1e-12
MAXITEM = 32

NC = 2
NS = 16
NW = NC * NS
LANES = 16
NVH = HID // LANES


def _posid_body(ids_ref, out_ref):
    ids = ids_ref[...]
    m = (ids != PAD).astype(jnp.int32)
    acc = m
    s = ids.shape[1]
    k = 1
    while k < s:
        shifted = jnp.concatenate(
            [jnp.zeros(ids.shape[:1] + (k,), jnp.int32), acc[:, :-k]], axis=1
        )
        acc = acc + shifted
        k *= 2
    out_ref[...] = acc * m + PAD


def _sc_body(tpw, chunk, word_hbm, pos_hbm, idw_hbm, idp_hbm, out_hbm,
             idw_all, idp_all, bw0, bp0, bw1, bp1, sem, wsem):
    wid = lax.axis_index("s") * NC + lax.axis_index("c")
    base = wid * tpw
    nchunks = tpw // chunk
    pltpu.sync_copy(idw_hbm.at[pl.ds(base, tpw)], idw_all)
    pltpu.sync_copy(idp_hbm.at[pl.ds(base, tpw)], idp_all)

    bufs = [(bw0, bp0), (bw1, bp1)]

    def issue(g):
        bw, bp = bufs[g % 2]
        sl = pl.ds(g * chunk, chunk)
        cw = pltpu.async_copy(word_hbm.at[idw_all.at[sl]], bw, sem)
        cp = pltpu.async_copy(pos_hbm.at[idp_all.at[sl]], bp, sem)
        return cw, cp

    pend = issue(0)
    wpend = [None, None]
    for g in range(nchunks):
        pend[0].wait()
        pend[1].wait()
        if g + 1 < nchunks:
            if wpend[(g + 1) % 2] is not None:
                wpend[(g + 1) % 2].wait()
                wpend[(g + 1) % 2] = None
            pend = issue(g + 1)
        bw, bp = bufs[g % 2]

        def add_token(t, _, bw=bw, bp=bp):
            for i in range(NVH):
                sl = pl.ds(i * LANES, LANES)
                bw[t, sl] = bw[t, sl] + bp[t, sl]
            return 0

        lax.fori_loop(0, chunk, add_token, 0)
        wpend[g % 2] = pltpu.async_copy(
            bw, out_hbm.at[pl.ds(base + g * chunk, chunk)], wsem)
    for w in wpend:
        if w is not None:
            w.wait()


def _ln_body(ids_ref, x_ref, item_ref, w_ref, b_ref, o_ref):
    x = x_ref[...]
    ids = ids_ref[...]
    onehot = (ids == lax.broadcasted_iota(
        jnp.int32, (ids.shape[0], MAXITEM), 1)).astype(jnp.float32)
    x = x + jnp.dot(onehot, item_ref[...], preferred_element_type=jnp.float32,
                    precision=lax.Precision.HIGHEST)
    mu = jnp.mean(x, axis=-1, keepdims=True)
    var = jnp.mean(jnp.square(x - mu), axis=-1, keepdims=True)
    o_ref[...] = (x - mu) / jnp.sqrt(var + EPS) * w_ref[...] + b_ref[...]


def kernel(input_ids, item_position_ids, word_embeddings, position_embeddings,
           item_position_embeddings, ln_weight, ln_bias):
    b, s = input_ids.shape
    n = b * s
    tpw = n // NW
    chunk = 32

    position_ids = pl.pallas_call(
        _posid_body,
        out_shape=jax.ShapeDtypeStruct((b, s), jnp.int32),
    )(input_ids.astype(jnp.int32))

    mesh = plsc.VectorSubcoreMesh(core_axis_name="c", subcore_axis_name="s")
    sc = pl.kernel(
        functools.partial(_sc_body, tpw, chunk),
        out_type=jax.ShapeDtypeStruct((n, HID), jnp.float32),
        mesh=mesh,
        scratch_types=[
            pltpu.VMEM((tpw,), jnp.int32),
            pltpu.VMEM((tpw,), jnp.int32),
            pltpu.VMEM((chunk, HID), jnp.float32),
            pltpu.VMEM((chunk, HID), jnp.float32),
            pltpu.VMEM((chunk, HID), jnp.float32),
            pltpu.VMEM((chunk, HID), jnp.float32),
            pltpu.SemaphoreType.DMA,
            pltpu.SemaphoreType.DMA,
        ],
    )
    summed = sc(
        word_embeddings,
        position_embeddings,
        input_ids.reshape(n).astype(jnp.int32),
        position_ids.reshape(n),
    )

    tb = 512
    nblk = n // tb
    out = pl.pallas_call(
        _ln_body,
        grid=(nblk,),
        in_specs=[
            pl.BlockSpec((tb, 1), lambda i: (i, 0)),
            pl.BlockSpec((tb, HID), lambda i: (i, 0)),
            pl.BlockSpec((MAXITEM, HID), lambda i: (0, 0)),
            pl.BlockSpec((HID,), lambda i: (0,)),
            pl.BlockSpec((HID,), lambda i: (0,)),
        ],
        out_specs=pl.BlockSpec((tb, HID), lambda i: (i, 0)),
        out_shape=jax.ShapeDtypeStruct((n, HID), jnp.float32),
    )(
        item_position_ids.reshape(n, 1).astype(jnp.int32),
        summed,
        item_position_embeddings,
        ln_weight,
        ln_bias,
    )
    return out.reshape(b, s, HID)
